# baseline (speedup 1.0000x reference)
import functools

import jax
import jax.numpy as jnp
from jax import lax
from jax.experimental import pallas as pl
from jax.experimental.pallas import tpu as pltpu
from jax.experimental.pallas import tpu_sc as plsc

_B, _HIS, _N, _S, _IN_DIM, _HID = 16, 13, 512, 3, 2, 128
_SN = _S * _N
_CKPTS = [3, 5, 7, 9, 11, 13]
_BPAIR = 4
_EPS = 1e-5

_NW = 32
_ROWS = _SN // _NW
_NCHUNK = 4


def _densify(dst_f, src_f, w_f, dst_b, src_b, w_b):
    e_fwd = dst_f.shape[0]
    e_bwd = dst_b.shape[0]
    cmax = max(e_fwd, e_bwd) // _NCHUNK
    mesh = plsc.VectorSubcoreMesh(core_axis_name="c", subcore_axis_name="s")
    zeros_band = jnp.zeros((_ROWS * _SN,), jnp.float32)

    @functools.partial(
        pl.kernel,
        mesh=mesh,
        compiler_params=pltpu.CompilerParams(needs_layout_passes=False),
        out_type=(jax.ShapeDtypeStruct((_SN * _SN,), jnp.float32),
                  jax.ShapeDtypeStruct((_SN * _SN,), jnp.float32)),
        scratch_types=[
            pltpu.VMEM((_ROWS * _SN,), jnp.float32),
            pltpu.VMEM((cmax,), jnp.int32),
            pltpu.VMEM((cmax,), jnp.int32),
            pltpu.VMEM((cmax,), jnp.float32),
        ],
    )
    def dens(zeros_h, dstf_h, srcf_h, wf_h, dstb_h, srcb_h, wb_h, af_h, ab_h,
             acc, dbuf, sbuf, wbuf):
        wid = lax.axis_index("s") * 2 + lax.axis_index("c")
        base = wid * _ROWS

        def run_phase(n_edges, dst_h, src_h, w_h, out_h):
            pltpu.sync_copy(zeros_h, acc)
            chunk = n_edges // _NCHUNK
            lanes = lax.iota(jnp.int32, 16)

            def chunk_body(k, carry):
                off = k * chunk
                pltpu.sync_copy(dst_h.at[pl.ds(off, chunk)], dbuf.at[pl.ds(0, chunk)])
                pltpu.sync_copy(src_h.at[pl.ds(off, chunk)], sbuf.at[pl.ds(0, chunk)])
                pltpu.sync_copy(w_h.at[pl.ds(off, chunk)], wbuf.at[pl.ds(0, chunk)])

                def one_vec(off16):
                    d = dbuf[pl.ds(off16, 16)]
                    m = (d >= base) & (d < base + _ROWS)

                    @pl.when(jnp.any(m))
                    def _():
                        s = sbuf[pl.ds(off16, 16)]
                        w = wbuf[pl.ds(off16, 16)]
                        flat = jnp.where(m, (d - base) * _SN + s, 0)
                        cnt = jnp.sum(m.astype(jnp.int32))

                        @pl.when(cnt == 1)
                        def _():
                            plsc.addupdate_scatter(
                                acc.at[pl.ds(0, _ROWS * _SN)], [flat], w,
                                mask=m)

                        @pl.when(cnt > 1)
                        def _():
                            for l in range(16):
                                plsc.addupdate_scatter(
                                    acc.at[pl.ds(0, _ROWS * _SN)], [flat], w,
                                    mask=m & (lanes == l))

                def vec_body(i, carry2):
                    for u in range(4):
                        one_vec(i * 64 + u * 16)
                    return carry2
                lax.fori_loop(0, chunk // 64, vec_body, carry)
                return carry
            lax.fori_loop(0, _NCHUNK, chunk_body, 0)
            pltpu.sync_copy(acc, out_h.at[pl.ds(base * _SN, _ROWS * _SN)])

        run_phase(e_fwd, dstf_h, srcf_h, wf_h, af_h)
        run_phase(e_bwd, dstb_h, srcb_h, wb_h, ab_h)

    return dens(zeros_band, dst_f, src_f, w_f, dst_b, src_b, w_b)


def _dot(a, b):
    return jnp.dot(a, b, preferred_element_type=jnp.float32)


def _fused_body(x0_ref, xs_ref, win_ref, bin_ref, af_ref, ab_ref,
                wra_ref, wrp_ref, bra_ref, brp_ref, wga_ref, wgp_ref,
                out_a_ref, out_p_ref, st_a, st_p, stats, sums):
    c = pl.program_id(0)
    b = pl.program_id(1)
    m_count = float(_B * _N * _HID)
    n_rounds = len(_CKPTS)

    @pl.when((b == 0) & (c == 0))
    def _():
        for si in range(2):
            stats[si, 0] = 0.0
            stats[si, 1] = 1.0
            sums[si, 0] = 0.0
            sums[si, 1] = 0.0

    @pl.when((b == 0) & (c > 0))
    def _():
        for si in range(2):
            mu = sums[si, 0] / m_count
            var = sums[si, 1] / m_count - mu * mu
            stats[si, 0] = mu
            stats[si, 1] = lax.rsqrt(var + _EPS)
            sums[si, 0] = 0.0
            sums[si, 1] = 0.0

    @pl.when(c < n_rounds)
    def _():
        for u in range(_BPAIR):
            bb = b * _BPAIR + u
            xsb = _dot(xs_ref[u, 0], win_ref[...]) + bin_ref[...]
            x0b = _dot(x0_ref[u], win_ref[...]) + bin_ref[...]
            sf_sh = _dot(af_ref[:, _N:], xsb)
            sb_sh = _dot(ab_ref[:, _N:], xsb)
            streams = ((0, st_a, wra_ref, bra_ref, wga_ref),
                       (1, st_p, wrp_ref, brp_ref, wgp_ref))
            for si, st, wr_ref, br_ref, wg_ref in streams:
                prev = jnp.where(c == 0, x0b,
                                 (st[bb] - stats[si, 0]) * stats[si, 1])
                pf = _dot(af_ref[:, :_N], prev) + sf_sh
                pb = _dot(ab_ref[:, :_N], prev) + sb_sh
                h1 = jnp.maximum(_dot(pf, wg_ref[0, 0]) + _dot(pb, wg_ref[0, 1]), 0.0)
                pf2 = _dot(af_ref[...], h1)
                pb2 = _dot(ab_ref[...], h1)
                h2 = jnp.maximum(_dot(pf2, wg_ref[1, 0]) + _dot(pb2, wg_ref[1, 1]), 0.0)
                gcn = h2[:_N] + h2[_N:2 * _N] + h2[2 * _N:]
                res = _dot(wr_ref[:, :_N], prev) + _dot(wr_ref[:, _N:], xsb) + br_ref[...]
                o = gcn + res
                st[bb] = o
                sums[si, 0] += jnp.sum(o)
                sums[si, 1] += jnp.sum(o * o)

    @pl.when(c == n_rounds)
    def _():
        for u in range(_BPAIR):
            bb = b * _BPAIR + u
            out_a_ref[u] = (st_a[bb] - stats[0, 0]) * stats[0, 1]
            out_p_ref[u] = (st_p[bb] - stats[1, 0]) * stats[1, 1]


def _fused(x0, xs_all, w_in, b_in, a_f, a_b, w_res_a, w_res_p,
           b_res_a, b_res_p, wg_a, wg_p):
    full = lambda *s: pl.BlockSpec(s, lambda c, b: (0,) * len(s))
    n_rounds = len(_CKPTS)
    nb = _B // _BPAIR
    return pl.pallas_call(
        _fused_body,
        grid=(n_rounds + 1, nb),
        in_specs=[pl.BlockSpec((_BPAIR, _N, _IN_DIM), lambda c, b: (b, 0, 0)),
                  pl.BlockSpec((_BPAIR, 1, 2 * _N, _IN_DIM),
                               lambda c, b: (b, jnp.minimum(c, n_rounds - 1), 0, 0)),
                  full(_IN_DIM, _HID), full(1, _HID),
                  full(_SN, _SN), full(_SN, _SN),
                  full(_N, _SN), full(_N, _SN),
                  full(_N, 1), full(_N, 1),
                  full(2, 2, _HID, _HID), full(2, 2, _HID, _HID)],
        out_specs=[pl.BlockSpec((_BPAIR, _N, _HID),
                                lambda c, b: (jnp.where(c == n_rounds, b, _B // _BPAIR), 0, 0)),
                   pl.BlockSpec((_BPAIR, _N, _HID),
                                lambda c, b: (jnp.where(c == n_rounds, b, _B // _BPAIR), 0, 0))],
        out_shape=[jax.ShapeDtypeStruct((_B + _BPAIR, _N, _HID), jnp.float32),
                   jax.ShapeDtypeStruct((_B + _BPAIR, _N, _HID), jnp.float32)],
        scratch_shapes=[pltpu.VMEM((_B, _N, _HID), jnp.float32),
                        pltpu.VMEM((_B, _N, _HID), jnp.float32),
                        pltpu.SMEM((2, 2), jnp.float32),
                        pltpu.SMEM((2, 2), jnp.float32)],
    )(x0, xs_all, w_in, b_in.reshape(1, _HID), a_f, a_b, w_res_a, w_res_p,
      b_res_a.reshape(_N, 1), b_res_p.reshape(_N, 1), wg_a, wg_p)


def kernel(inputs, edge_src_fwd, edge_dst_fwd, edge_w_fwd,
           edge_src_bwd, edge_dst_bwd, edge_w_bwd,
           W_in, b_in, W_res_adj, b_res_adj, W_res_pea, b_res_pea,
           W_gcn_adj, W_gcn_pea):
    a_f, a_b = _densify(edge_dst_fwd, edge_src_fwd, edge_w_fwd,
                        edge_dst_bwd, edge_src_bwd, edge_w_bwd)
    a_f = a_f.reshape(_SN, _SN)
    a_b = a_b.reshape(_SN, _SN)

    x0 = inputs[:, 0]
    xs_all = inputs[:, 1:].reshape(_B, len(_CKPTS), 2 * _N, _IN_DIM)
    out_a, out_p = _fused(x0, xs_all, W_in, b_in, a_f, a_b, W_res_adj,
                          W_res_pea, b_res_adj, b_res_pea, W_gcn_adj, W_gcn_pea)
    return out_a[:_B, None], out_p[:_B, None]


# speedup vs baseline: 1.2223x; 1.2223x over previous
import functools

import jax
import jax.numpy as jnp
from jax import lax
from jax.experimental import pallas as pl
from jax.experimental.pallas import tpu as pltpu
from jax.experimental.pallas import tpu_sc as plsc

_B, _HIS, _N, _S, _IN_DIM, _HID = 16, 13, 512, 3, 2, 128
_SN = _S * _N
_CKPTS = [3, 5, 7, 9, 11, 13]
_BPAIR = 2
_EPS = 1e-5

_NW = 32
_ROWS = _SN // _NW
_NCHUNK = 4


def _densify(dst_f, src_f, w_f, dst_b, src_b, w_b):
    e_fwd = dst_f.shape[0]
    e_bwd = dst_b.shape[0]
    cmax = max(e_fwd, e_bwd) // _NCHUNK
    mesh = plsc.VectorSubcoreMesh(core_axis_name="c", subcore_axis_name="s")
    zeros_band = jnp.zeros((_ROWS * _SN,), jnp.float32)

    @functools.partial(
        pl.kernel,
        mesh=mesh,
        compiler_params=pltpu.CompilerParams(needs_layout_passes=False),
        out_type=(jax.ShapeDtypeStruct((_SN * _SN,), jnp.float32),
                  jax.ShapeDtypeStruct((_SN * _SN,), jnp.float32)),
        scratch_types=[
            pltpu.VMEM((_ROWS * _SN,), jnp.float32),
            pltpu.VMEM((cmax,), jnp.int32),
            pltpu.VMEM((cmax,), jnp.int32),
            pltpu.VMEM((cmax,), jnp.float32),
        ],
    )
    def dens(zeros_h, dstf_h, srcf_h, wf_h, dstb_h, srcb_h, wb_h, af_h, ab_h,
             acc, dbuf, sbuf, wbuf):
        wid = lax.axis_index("s") * 2 + lax.axis_index("c")
        base = wid * _ROWS

        def run_phase(n_edges, dst_h, src_h, w_h, out_h):
            pltpu.sync_copy(zeros_h, acc)
            chunk = n_edges // _NCHUNK
            lanes = lax.iota(jnp.int32, 16)

            def chunk_body(k, carry):
                off = k * chunk
                pltpu.sync_copy(dst_h.at[pl.ds(off, chunk)], dbuf.at[pl.ds(0, chunk)])
                pltpu.sync_copy(src_h.at[pl.ds(off, chunk)], sbuf.at[pl.ds(0, chunk)])
                pltpu.sync_copy(w_h.at[pl.ds(off, chunk)], wbuf.at[pl.ds(0, chunk)])

                def one_vec(off16):
                    d = dbuf[pl.ds(off16, 16)]
                    m = (d >= base) & (d < base + _ROWS)

                    @pl.when(jnp.any(m))
                    def _():
                        s = sbuf[pl.ds(off16, 16)]
                        w = wbuf[pl.ds(off16, 16)]
                        flat = jnp.where(m, (d - base) * _SN + s, 0)
                        cnt = jnp.sum(m.astype(jnp.int32))

                        @pl.when(cnt == 1)
                        def _():
                            plsc.addupdate_scatter(
                                acc.at[pl.ds(0, _ROWS * _SN)], [flat], w,
                                mask=m)

                        @pl.when(cnt > 1)
                        def _():
                            for l in range(16):
                                plsc.addupdate_scatter(
                                    acc.at[pl.ds(0, _ROWS * _SN)], [flat], w,
                                    mask=m & (lanes == l))

                def vec_body(i, carry2):
                    for u in range(4):
                        one_vec(i * 64 + u * 16)
                    return carry2
                lax.fori_loop(0, chunk // 64, vec_body, carry)
                return carry
            lax.fori_loop(0, _NCHUNK, chunk_body, 0)
            pltpu.sync_copy(acc, out_h.at[pl.ds(base * _SN, _ROWS * _SN)])

        run_phase(e_fwd, dstf_h, srcf_h, wf_h, af_h)
        run_phase(e_bwd, dstb_h, srcb_h, wb_h, ab_h)

    return dens(zeros_band, dst_f, src_f, w_f, dst_b, src_b, w_b)


def _dot(a, b):
    return jnp.dot(a, b, preferred_element_type=jnp.float32)


def _fused_body(x0_ref, xs_ref, win_ref, bin_ref, af_ref, ab_ref,
                wra_ref, wrp_ref, bra_ref, brp_ref, wga_ref, wgp_ref,
                out_a_ref, out_p_ref, st_a, st_p, stats, sums):
    c = pl.program_id(0)
    b = pl.program_id(1)
    m_count = float(_B * _N * _HID)
    n_rounds = len(_CKPTS)

    @pl.when((b == 0) & (c == 0))
    def _():
        for si in range(2):
            stats[si, 0] = 0.0
            stats[si, 1] = 1.0
            sums[si, 0] = 0.0
            sums[si, 1] = 0.0

    @pl.when((b == 0) & (c > 0))
    def _():
        for si in range(2):
            mu = sums[si, 0] / m_count
            var = sums[si, 1] / m_count - mu * mu
            stats[si, 0] = mu
            stats[si, 1] = lax.rsqrt(var + _EPS)
            sums[si, 0] = 0.0
            sums[si, 1] = 0.0

    @pl.when(c < n_rounds)
    def _():
        for u in range(_BPAIR):
            bb = b * _BPAIR + u
            xsb = _dot(xs_ref[u, 0], win_ref[...]) + bin_ref[...]
            x0b = _dot(x0_ref[u], win_ref[...]) + bin_ref[...]
            sf_sh = _dot(af_ref[:, _N:], xsb)
            sb_sh = _dot(ab_ref[:, _N:], xsb)
            streams = ((0, st_a, wra_ref, bra_ref, wga_ref),
                       (1, st_p, wrp_ref, brp_ref, wgp_ref))
            for si, st, wr_ref, br_ref, wg_ref in streams:
                prev = jnp.where(c == 0, x0b,
                                 (st[bb] - stats[si, 0]) * stats[si, 1])
                pf = _dot(af_ref[:, :_N], prev) + sf_sh
                pb = _dot(ab_ref[:, :_N], prev) + sb_sh
                h1 = jnp.maximum(_dot(pf, wg_ref[0, 0]) + _dot(pb, wg_ref[0, 1]), 0.0)
                pf2 = _dot(af_ref[...], h1)
                pb2 = _dot(ab_ref[...], h1)
                h2 = jnp.maximum(_dot(pf2, wg_ref[1, 0]) + _dot(pb2, wg_ref[1, 1]), 0.0)
                gcn = h2[:_N] + h2[_N:2 * _N] + h2[2 * _N:]
                res = _dot(wr_ref[:, :_N], prev) + _dot(wr_ref[:, _N:], xsb) + br_ref[...]
                o = gcn + res
                st[bb] = o
                sums[si, 0] += jnp.sum(o)
                sums[si, 1] += jnp.sum(o * o)

    @pl.when(c == n_rounds)
    def _():
        for u in range(_BPAIR):
            bb = b * _BPAIR + u
            out_a_ref[u] = (st_a[bb] - stats[0, 0]) * stats[0, 1]
            out_p_ref[u] = (st_p[bb] - stats[1, 0]) * stats[1, 1]


def _fused(x0, xs_all, w_in, b_in, a_f, a_b, w_res_a, w_res_p,
           b_res_a, b_res_p, wg_a, wg_p):
    full = lambda *s: pl.BlockSpec(s, lambda c, b: (0,) * len(s))
    n_rounds = len(_CKPTS)
    nb = _B // _BPAIR
    return pl.pallas_call(
        _fused_body,
        grid=(n_rounds + 1, nb),
        in_specs=[pl.BlockSpec((_BPAIR, _N, _IN_DIM), lambda c, b: (b, 0, 0)),
                  pl.BlockSpec((_BPAIR, 1, 2 * _N, _IN_DIM),
                               lambda c, b: (b, jnp.minimum(c, n_rounds - 1), 0, 0)),
                  full(_IN_DIM, _HID), full(1, _HID),
                  full(_SN, _SN), full(_SN, _SN),
                  full(_N, _SN), full(_N, _SN),
                  full(_N, 1), full(_N, 1),
                  full(2, 2, _HID, _HID), full(2, 2, _HID, _HID)],
        out_specs=[pl.BlockSpec((_BPAIR, _N, _HID),
                                lambda c, b: (jnp.where(c == n_rounds, b, _B // _BPAIR), 0, 0)),
                   pl.BlockSpec((_BPAIR, _N, _HID),
                                lambda c, b: (jnp.where(c == n_rounds, b, _B // _BPAIR), 0, 0))],
        out_shape=[jax.ShapeDtypeStruct((_B + _BPAIR, _N, _HID), jnp.float32),
                   jax.ShapeDtypeStruct((_B + _BPAIR, _N, _HID), jnp.float32)],
        scratch_shapes=[pltpu.VMEM((_B, _N, _HID), jnp.float32),
                        pltpu.VMEM((_B, _N, _HID), jnp.float32),
                        pltpu.SMEM((2, 2), jnp.float32),
                        pltpu.SMEM((2, 2), jnp.float32)],
    )(x0, xs_all, w_in, b_in.reshape(1, _HID), a_f, a_b, w_res_a, w_res_p,
      b_res_a.reshape(_N, 1), b_res_p.reshape(_N, 1), wg_a, wg_p)


def kernel(inputs, edge_src_fwd, edge_dst_fwd, edge_w_fwd,
           edge_src_bwd, edge_dst_bwd, edge_w_bwd,
           W_in, b_in, W_res_adj, b_res_adj, W_res_pea, b_res_pea,
           W_gcn_adj, W_gcn_pea):
    a_f, a_b = _densify(edge_dst_fwd, edge_src_fwd, edge_w_fwd,
                        edge_dst_bwd, edge_src_bwd, edge_w_bwd)
    a_f = a_f.reshape(_SN, _SN)
    a_b = a_b.reshape(_SN, _SN)

    x0 = inputs[:, 0]
    xs_all = inputs[:, 1:].reshape(_B, len(_CKPTS), 2 * _N, _IN_DIM)
    out_a, out_p = _fused(x0, xs_all, W_in, b_in, a_f, a_b, W_res_adj,
                          W_res_pea, b_res_adj, b_res_pea, W_gcn_adj, W_gcn_pea)
    return out_a[:_B, None], out_p[:_B, None]
